# Initial kernel scaffold; baseline (speedup 1.0000x reference)
import jax
import jax.numpy as jnp
from jax.experimental import pallas as pl


def kernel(x1, x2, x3, edge_index1, edge_index2, edge_index3, conv1_weight, conv1_bias, conv2_weight, conv2_bias, conv3_weight, conv3_bias, sa_fc1_weight, sa_fc1_bias, sa_fc2_weight, sa_fc2_bias, sa_fc3_weight, sa_fc3_bias, fc1_weight, fc1_bias, fc2_weight, fc2_bias):
    raise NotImplementedError("write your pallas kernel here")


# speedup vs baseline: 16.6770x; 16.6770x over previous
import functools

import jax
import jax.numpy as jnp
from jax import lax
from jax.experimental import pallas as pl
from jax.experimental.pallas import tpu as pltpu
from jax.experimental.pallas import tpu_sc as plsc

N = 10000
E = 320000
D = 128
NPAD = 10240
EPAD = 327680
NT = 16
ROWS_PER_TILE = NPAD // NT
CHUNK = 128

@functools.cache
def _build_deg_sc():
    mesh = plsc.VectorSubcoreMesh(core_axis_name="c", subcore_axis_name="s")
    return pl.kernel(
        _deg_sc_body,
        out_type=jax.ShapeDtypeStruct((3, 2, NPAD), jnp.float32),
        mesh=mesh,
        scratch_types=[
            pltpu.VMEM_SHARED((NPAD,), jnp.float32),
            pltpu.VMEM((CHUNK,), jnp.int32),
            pltpu.VMEM((CHUNK,), jnp.float32),
        ],
    )


def _deg_sc_body(col1, col2, col3, zeros1, ones, out, deg_sh, cidx_v, ones_v):
    c = lax.axis_index("c")
    s = lax.axis_index("s")
    pltpu.sync_copy(ones, ones_v)
    e_core = EPAD // 2
    e_tile = e_core // NT
    n_ch = e_tile // CHUNK
    for i, col in enumerate((col1, col2, col3)):
        pltpu.sync_copy(zeros1.at[pl.ds(s * ROWS_PER_TILE, ROWS_PER_TILE)],
                        deg_sh.at[pl.ds(s * ROWS_PER_TILE, ROWS_PER_TILE)])
        plsc.subcore_barrier()

        def body(j, _, col=col):
            base = c * e_core + s * e_tile + j * CHUNK
            pltpu.sync_copy(col.at[pl.ds(base, CHUNK)], cidx_v)
            pltpu.sync_copy(ones_v, deg_sh.at[cidx_v], add=True)
            return 0

        lax.fori_loop(0, n_ch, body, 0)
        plsc.subcore_barrier()
        pltpu.sync_copy(deg_sh.at[pl.ds(s * ROWS_PER_TILE, ROWS_PER_TILE)],
                        out.at[i, c, pl.ds(s * ROWS_PER_TILE, ROWS_PER_TILE)])
        plsc.subcore_barrier()


@functools.cache
def _build_scatter_sc():
    mesh = plsc.VectorSubcoreMesh(core_axis_name="c", subcore_axis_name="s")
    return pl.kernel(
        _scatter_sc_body,
        out_type=jax.ShapeDtypeStruct((3, 2, NPAD, D), jnp.float32),
        mesh=mesh,
        scratch_types=[
            pltpu.VMEM_SHARED((NPAD, D), jnp.float32),
            pltpu.VMEM((CHUNK,), jnp.int32),
            pltpu.VMEM((CHUNK,), jnp.int32),
            pltpu.VMEM((CHUNK, D), jnp.float32),
            pltpu.SemaphoreType.DMA,
        ],
    )


def _scatter_sc_body(g, r1, c1, r2, c2, r3, c3, zeros2, out,
                     acc_sh, ridx_v, cidx_v, rows_v, sem):
    c = lax.axis_index("c")
    s = lax.axis_index("s")
    e_core = EPAD // 2
    e_tile = e_core // NT
    n_ch = e_tile // CHUNK
    rsl = pl.ds(s * ROWS_PER_TILE, ROWS_PER_TILE)
    for i, (rr, cc) in enumerate(((r1, c1), (r2, c2), (r3, c3))):
        pltpu.sync_copy(zeros2.at[rsl], acc_sh.at[rsl])
        plsc.subcore_barrier()

        def body(j, _, i=i, rr=rr, cc=cc):
            base = c * e_core + s * e_tile + j * CHUNK
            pltpu.sync_copy(rr.at[pl.ds(base, CHUNK)], ridx_v)
            pltpu.sync_copy(cc.at[pl.ds(base, CHUNK)], cidx_v)
            pltpu.async_copy(g.at[i].at[ridx_v], rows_v, sem).wait()
            pltpu.sync_copy(rows_v, acc_sh.at[cidx_v], add=True)
            return 0

        lax.fori_loop(0, n_ch, body, 0)
        plsc.subcore_barrier()
        pltpu.sync_copy(acc_sh.at[rsl], out.at[i, c, rsl])
        plsc.subcore_barrier()


def _dense_body(x_ref, w_ref, deg_ref, g_ref, dinv_ref):
    h = jnp.dot(x_ref[0], w_ref[0], preferred_element_type=jnp.float32)
    deg = deg_ref[0, 0] + deg_ref[0, 1] + 1.0
    dinv = lax.rsqrt(deg)
    g_ref[0] = h * dinv[:, None]
    dinv_ref[0, 0] = dinv


_RB = 256


def _dense_tc(xs, ws, deg):
    grid = (3, NPAD // _RB)
    return pl.pallas_call(
        _dense_body,
        grid=grid,
        in_specs=[
            pl.BlockSpec((1, _RB, D), lambda i, j: (i, j, 0)),
            pl.BlockSpec((1, D, D), lambda i, j: (i, 0, 0)),
            pl.BlockSpec((1, 2, _RB), lambda i, j: (i, 0, j)),
        ],
        out_specs=[
            pl.BlockSpec((1, _RB, D), lambda i, j: (i, j, 0)),
            pl.BlockSpec((1, 1, _RB), lambda i, j: (i, 0, j)),
        ],
        out_shape=[
            jax.ShapeDtypeStruct((3, NPAD, D), jnp.float32),
            jax.ShapeDtypeStruct((3, 1, NPAD), jnp.float32),
        ],
    )(xs, ws, deg)


def _reduce_body(acc_ref, g_ref, dinv_ref, b_ref, v_ref):
    j = pl.program_id(1)
    rows = acc_ref[0, 0] + acc_ref[0, 1] + g_ref[0]
    o = rows * dinv_ref[0, 0][:, None] + b_ref[0, 0]
    o = jnp.maximum(o, 0.0)
    gid = j * _RB + lax.broadcasted_iota(jnp.int32, (_RB, 1), 0)
    o = jnp.where(gid < N, o, 0.0)
    part = jnp.sum(o, axis=0, keepdims=True)

    @pl.when(j == 0)
    def _():
        v_ref[0] = part

    @pl.when(j > 0)
    def _():
        v_ref[0] = v_ref[0] + part


def _reduce_tc(acc, g, dinv, bs):
    grid = (3, NPAD // _RB)
    return pl.pallas_call(
        _reduce_body,
        grid=grid,
        in_specs=[
            pl.BlockSpec((1, 2, _RB, D), lambda i, j: (i, 0, j, 0)),
            pl.BlockSpec((1, _RB, D), lambda i, j: (i, j, 0)),
            pl.BlockSpec((1, 1, _RB), lambda i, j: (i, 0, j)),
            pl.BlockSpec((1, 1, D), lambda i, j: (i, 0, 0)),
        ],
        out_specs=pl.BlockSpec((1, 1, D), lambda i, j: (i, 0, 0)),
        out_shape=jax.ShapeDtypeStruct((3, 1, D), jnp.float32),
    )(acc, g, dinv, bs)


def _head_body(v_ref, w1_ref, b1_ref, w2_ref, b2_ref, w3_ref, b3_ref,
               fw1_ref, fb1_ref, fw2_ref, fb2_ref, gam_ref, bet_ref):
    f32 = jnp.float32
    v = v_ref[...]
    dg = lambda a, b: lax.dot_general(a, b, (((1,), (1,)), ((), ())),
                                      preferred_element_type=f32)
    t = jnp.zeros((1, D), f32)
    for k in range(3):
        t = t + dg(v[k:k + 1, :], w1_ref[:, D * k:D * (k + 1)])
    x_out = jnp.maximum(t + b1_ref[...][None, :], 0.0)
    x2 = dg(x_out, w2_ref[...]) + b2_ref[...][None, :]
    z = dg(jnp.maximum(x2, 0.0), w3_ref[...]) + b3_ref[...][None, :]
    z = z - jnp.max(z, axis=-1, keepdims=True)
    ez = jnp.exp(z)
    a = ez / jnp.sum(ez, axis=-1, keepdims=True)
    x = lax.dot_general(a, v, (((1,), (0,)), ((), ())),
                        preferred_element_type=f32)
    gam_ref[...] = jnp.tanh(dg(x, fw1_ref[...]) + fb1_ref[...][None, :])
    bet_ref[...] = jnp.tanh(dg(x, fw2_ref[...]) + fb2_ref[...][None, :])


def _head_tc(v, w1, b1, w2, b2, w3, b3, fw1, fb1, fw2, fb2):
    return pl.pallas_call(
        _head_body,
        out_shape=[
            jax.ShapeDtypeStruct((1, D), jnp.float32),
            jax.ShapeDtypeStruct((1, D), jnp.float32),
        ],
    )(v, w1, b1, w2, b2, w3, b3, fw1, fb1, fw2, fb2)


def kernel(x1, x2, x3, edge_index1, edge_index2, edge_index3,
           conv1_weight, conv1_bias, conv2_weight, conv2_bias,
           conv3_weight, conv3_bias,
           sa_fc1_weight, sa_fc1_bias, sa_fc2_weight, sa_fc2_bias,
           sa_fc3_weight, sa_fc3_bias,
           fc1_weight, fc1_bias, fc2_weight, fc2_bias):
    pad = EPAD - E
    pad_idx = N + (jnp.arange(pad, dtype=jnp.int32) % (NPAD - N))
    rows, cols = [], []
    for e in (edge_index1, edge_index2, edge_index3):
        rows.append(jnp.concatenate([e[0], pad_idx]))
        cols.append(jnp.concatenate([e[1], pad_idx]))
    zeros1 = jnp.zeros((NPAD,), jnp.float32)
    zeros2 = jnp.zeros((NPAD, D), jnp.float32)
    ones = jnp.ones((CHUNK,), jnp.float32)

    deg = _build_deg_sc()(cols[0], cols[1], cols[2], zeros1, ones)

    xs = jnp.zeros((3, NPAD, D), jnp.float32).at[:, :N, :].set(
        jnp.stack([x1, x2, x3]))
    ws = jnp.stack([conv1_weight, conv2_weight, conv3_weight])
    g, dinv = _dense_tc(xs, ws, deg)

    acc = _build_scatter_sc()(g, rows[0], cols[0], rows[1], cols[1],
                              rows[2], cols[2], zeros2)

    bs = jnp.stack([conv1_bias, conv2_bias, conv3_bias]).reshape(3, 1, D)
    v = _reduce_tc(acc, g, dinv, bs)

    gam, bet = _head_tc(v.reshape(3, D),
                        sa_fc1_weight, sa_fc1_bias,
                        sa_fc2_weight, sa_fc2_bias,
                        sa_fc3_weight, sa_fc3_bias,
                        fc1_weight, fc1_bias, fc2_weight, fc2_bias)
    return gam.reshape(D), bet.reshape(D)


# speedup vs baseline: 29.0447x; 1.7416x over previous
import functools

import jax
import jax.numpy as jnp
from jax import lax
from jax.experimental import pallas as pl
from jax.experimental.pallas import tpu as pltpu
from jax.experimental.pallas import tpu_sc as plsc

N = 10000
E = 320000
D = 128
NPAD = 10240
EPAD = 327680
NT = 16
ROWS_PER_TILE = NPAD // NT
CHUNK = 128

@functools.cache
def _build_deg_sc():
    mesh = plsc.VectorSubcoreMesh(core_axis_name="c", subcore_axis_name="s")
    return pl.kernel(
        _deg_sc_body,
        out_type=jax.ShapeDtypeStruct((3, 2, NPAD), jnp.float32),
        mesh=mesh,
        scratch_types=[
            pltpu.VMEM_SHARED((NPAD,), jnp.float32),
            pltpu.VMEM((_DG, CHUNK), jnp.int32),
            pltpu.VMEM((CHUNK,), jnp.float32),
        ],
    )


_DG = 8


def _deg_sc_body(col1, col2, col3, zeros1, ones, out, deg_sh, cidx_b, ones_v):
    c = lax.axis_index("c")
    s = lax.axis_index("s")
    pltpu.sync_copy(ones, ones_v)
    e_core = EPAD // 2
    e_tile = e_core // NT
    n_grp = e_tile // (CHUNK * _DG)
    for i, col in enumerate((col1, col2, col3)):
        pltpu.sync_copy(zeros1.at[pl.ds(s * ROWS_PER_TILE, ROWS_PER_TILE)],
                        deg_sh.at[pl.ds(s * ROWS_PER_TILE, ROWS_PER_TILE)])
        plsc.subcore_barrier()

        def body(t, _, col=col):
            crow = pl.multiple_of(
                (c * e_core + s * e_tile) // CHUNK + t * _DG, 8)
            pltpu.sync_copy(col.at[pl.ds(crow, _DG)], cidx_b)
            for m in range(_DG):
                pltpu.sync_copy(ones_v, deg_sh.at[cidx_b.at[m]], add=True)
            return 0

        lax.fori_loop(0, n_grp, body, 0)
        plsc.subcore_barrier()
        pltpu.sync_copy(deg_sh.at[pl.ds(s * ROWS_PER_TILE, ROWS_PER_TILE)],
                        out.at[i, c, pl.ds(s * ROWS_PER_TILE, ROWS_PER_TILE)])
        plsc.subcore_barrier()


_SG = 8
_SB = 2


@functools.cache
def _build_scatter_sc():
    mesh = plsc.VectorSubcoreMesh(core_axis_name="c", subcore_axis_name="s")
    return pl.kernel(
        _scatter_sc_body,
        out_type=jax.ShapeDtypeStruct((3, 2, NPAD, D), jnp.float32),
        mesh=mesh,
        scratch_types=[
            pltpu.VMEM_SHARED((NPAD, D), jnp.float32),
            pltpu.VMEM((_SG, CHUNK), jnp.int32),
            pltpu.VMEM((_SG, CHUNK), jnp.int32),
            [pltpu.VMEM((CHUNK, D), jnp.float32) for _ in range(_SB)],
            pltpu.SemaphoreType.DMA,
        ],
    )


def _scatter_sc_body(g, r1, c1, r2, c2, r3, c3, zeros2, out,
                     acc_sh, ridx_b, cidx_b, rows_bufs, sem):
    c = lax.axis_index("c")
    s = lax.axis_index("s")
    e_core = EPAD // 2
    e_tile = e_core // NT
    n_grp = e_tile // (CHUNK * _SG)
    rsl = pl.ds(s * ROWS_PER_TILE, ROWS_PER_TILE)
    for i, (rr, cc) in enumerate(((r1, c1), (r2, c2), (r3, c3))):
        pltpu.sync_copy(zeros2.at[rsl], acc_sh.at[rsl])
        plsc.subcore_barrier()

        def body(t, _, i=i, rr=rr, cc=cc):
            crow = pl.multiple_of(
                (c * e_core + s * e_tile) // CHUNK + t * _SG, 8)
            pltpu.sync_copy(rr.at[pl.ds(crow, _SG)], ridx_b)
            pltpu.sync_copy(cc.at[pl.ds(crow, _SG)], cidx_b)
            descs = [
                pltpu.async_copy(g.at[i].at[ridx_b.at[m]], rows_bufs[m], sem)
                for m in range(_SB)
            ]
            for m in range(_SG):
                descs[m % _SB].wait()
                pltpu.sync_copy(rows_bufs[m % _SB], acc_sh.at[cidx_b.at[m]],
                                add=True)
                if m + _SB < _SG:
                    descs[m % _SB] = pltpu.async_copy(
                        g.at[i].at[ridx_b.at[m + _SB]], rows_bufs[m % _SB], sem)
            return 0

        lax.fori_loop(0, n_grp, body, 0)
        plsc.subcore_barrier()
        pltpu.sync_copy(acc_sh.at[rsl], out.at[i, c, rsl])
        plsc.subcore_barrier()


def _dense_body(x_ref, w_ref, deg_ref, g_ref, dinv_ref):
    h = jnp.dot(x_ref[0], w_ref[0], preferred_element_type=jnp.float32)
    deg = deg_ref[0, 0] + deg_ref[0, 1] + 1.0
    dinv = lax.rsqrt(deg)
    g_ref[0] = h * dinv[:, None]
    dinv_ref[0, 0] = dinv


_RB = 256


def _dense_tc(xs, ws, deg):
    grid = (3, NPAD // _RB)
    return pl.pallas_call(
        _dense_body,
        grid=grid,
        in_specs=[
            pl.BlockSpec((1, _RB, D), lambda i, j: (i, j, 0)),
            pl.BlockSpec((1, D, D), lambda i, j: (i, 0, 0)),
            pl.BlockSpec((1, 2, _RB), lambda i, j: (i, 0, j)),
        ],
        out_specs=[
            pl.BlockSpec((1, _RB, D), lambda i, j: (i, j, 0)),
            pl.BlockSpec((1, 1, _RB), lambda i, j: (i, 0, j)),
        ],
        out_shape=[
            jax.ShapeDtypeStruct((3, NPAD, D), jnp.float32),
            jax.ShapeDtypeStruct((3, 1, NPAD), jnp.float32),
        ],
    )(xs, ws, deg)


def _reduce_body(acc_ref, g_ref, dinv_ref, b_ref, v_ref):
    j = pl.program_id(1)
    rows = acc_ref[0, 0] + acc_ref[0, 1] + g_ref[0]
    o = rows * dinv_ref[0, 0][:, None] + b_ref[0, 0]
    o = jnp.maximum(o, 0.0)
    gid = j * _RB + lax.broadcasted_iota(jnp.int32, (_RB, 1), 0)
    o = jnp.where(gid < N, o, 0.0)
    part = jnp.sum(o, axis=0, keepdims=True)

    @pl.when(j == 0)
    def _():
        v_ref[0] = part

    @pl.when(j > 0)
    def _():
        v_ref[0] = v_ref[0] + part


def _reduce_tc(acc, g, dinv, bs):
    grid = (3, NPAD // _RB)
    return pl.pallas_call(
        _reduce_body,
        grid=grid,
        in_specs=[
            pl.BlockSpec((1, 2, _RB, D), lambda i, j: (i, 0, j, 0)),
            pl.BlockSpec((1, _RB, D), lambda i, j: (i, j, 0)),
            pl.BlockSpec((1, 1, _RB), lambda i, j: (i, 0, j)),
            pl.BlockSpec((1, 1, D), lambda i, j: (i, 0, 0)),
        ],
        out_specs=pl.BlockSpec((1, 1, D), lambda i, j: (i, 0, 0)),
        out_shape=jax.ShapeDtypeStruct((3, 1, D), jnp.float32),
    )(acc, g, dinv, bs)


def _head_body(v_ref, w1_ref, b1_ref, w2_ref, b2_ref, w3_ref, b3_ref,
               fw1_ref, fb1_ref, fw2_ref, fb2_ref, gam_ref, bet_ref):
    f32 = jnp.float32
    v = v_ref[...]
    dg = lambda a, b: lax.dot_general(a, b, (((1,), (1,)), ((), ())),
                                      preferred_element_type=f32)
    t = jnp.zeros((1, D), f32)
    for k in range(3):
        t = t + dg(v[k:k + 1, :], w1_ref[:, D * k:D * (k + 1)])
    x_out = jnp.maximum(t + b1_ref[...][None, :], 0.0)
    x2 = dg(x_out, w2_ref[...]) + b2_ref[...][None, :]
    z = dg(jnp.maximum(x2, 0.0), w3_ref[...]) + b3_ref[...][None, :]
    z = z - jnp.max(z, axis=-1, keepdims=True)
    ez = jnp.exp(z)
    a = ez / jnp.sum(ez, axis=-1, keepdims=True)
    x = lax.dot_general(a, v, (((1,), (0,)), ((), ())),
                        preferred_element_type=f32)
    gam_ref[...] = jnp.tanh(dg(x, fw1_ref[...]) + fb1_ref[...][None, :])
    bet_ref[...] = jnp.tanh(dg(x, fw2_ref[...]) + fb2_ref[...][None, :])


def _head_tc(v, w1, b1, w2, b2, w3, b3, fw1, fb1, fw2, fb2):
    return pl.pallas_call(
        _head_body,
        out_shape=[
            jax.ShapeDtypeStruct((1, D), jnp.float32),
            jax.ShapeDtypeStruct((1, D), jnp.float32),
        ],
    )(v, w1, b1, w2, b2, w3, b3, fw1, fb1, fw2, fb2)


def kernel(x1, x2, x3, edge_index1, edge_index2, edge_index3,
           conv1_weight, conv1_bias, conv2_weight, conv2_bias,
           conv3_weight, conv3_bias,
           sa_fc1_weight, sa_fc1_bias, sa_fc2_weight, sa_fc2_bias,
           sa_fc3_weight, sa_fc3_bias,
           fc1_weight, fc1_bias, fc2_weight, fc2_bias):
    pad = EPAD - E
    pad_idx = N + (jnp.arange(pad, dtype=jnp.int32) % (NPAD - N))
    rows, cols = [], []
    for e in (edge_index1, edge_index2, edge_index3):
        rows.append(jnp.concatenate([e[0], pad_idx]).reshape(-1, CHUNK))
        cols.append(jnp.concatenate([e[1], pad_idx]).reshape(-1, CHUNK))
    zeros1 = jnp.zeros((NPAD,), jnp.float32)
    zeros2 = jnp.zeros((NPAD, D), jnp.float32)
    ones = jnp.ones((CHUNK,), jnp.float32)

    deg = _build_deg_sc()(cols[0], cols[1], cols[2], zeros1, ones)

    xs = jnp.zeros((3, NPAD, D), jnp.float32).at[:, :N, :].set(
        jnp.stack([x1, x2, x3]))
    ws = jnp.stack([conv1_weight, conv2_weight, conv3_weight])
    g, dinv = _dense_tc(xs, ws, deg)

    acc = _build_scatter_sc()(g, rows[0], cols[0], rows[1], cols[1],
                              rows[2], cols[2], zeros2)

    bs = jnp.stack([conv1_bias, conv2_bias, conv3_bias]).reshape(3, 1, D)
    v = _reduce_tc(acc, g, dinv, bs)

    gam, bet = _head_tc(v.reshape(3, D),
                        sa_fc1_weight, sa_fc1_bias,
                        sa_fc2_weight, sa_fc2_bias,
                        sa_fc3_weight, sa_fc3_bias,
                        fc1_weight, fc1_bias, fc2_weight, fc2_bias)
    return gam.reshape(D), bet.reshape(D)
